# baseline (speedup 1.0000x reference)
import functools

import jax
import jax.numpy as jnp
import numpy as np
from jax import lax
from jax.experimental import pallas as pl
from jax.experimental.pallas import tpu as pltpu
from jax.experimental.pallas import tpu_sc as plsc

VOCAB = 50257
NH = 2
EPS = 1e-6
TWO_PI = np.float32(2.0 * np.pi)

_NC = 2
_NS = 16
_NW = _NC * _NS

_VROWS = 512
_VP = _VROWS * 128
_VROWS_PER_S = _VROWS // _NS

_PROWS = 256
_PROWS_PER_W = _PROWS // _NW

_HEAD = 16


def _decode_block(emb_ref, pinv_ref, logv_ref, out_ref):
    x = emb_ref[...].reshape(-1, emb_ref.shape[2])
    w = pinv_ref[...]
    sincos = jnp.dot(x, w, preferred_element_type=jnp.float32)
    sin_part = sincos[:, :NH]
    cos_part = sincos[:, NH:]
    phase = jnp.arctan2(sin_part, cos_part)
    n = x.shape[0]
    pos = (lax.broadcasted_iota(jnp.int32, (n, 1), 0) % _HEAD).astype(jnp.float32)
    posc = jnp.maximum(pos, jnp.float32(EPS))
    f1 = phase[:, 0:1] / (TWO_PI * posc)
    f2 = phase[:, 1:2] / ((TWO_PI * np.float32(2.0)) * posc)
    freq_norm = (f1 + f2) * jnp.float32(1.0 / NH)
    freq_norm = jnp.clip(freq_norm, jnp.float32(0.0), jnp.float32(1.0))
    log_v = logv_ref[0]
    rank_est = jnp.exp(freq_norm * log_v)
    idx = jnp.ceil(rank_est - jnp.float32(1.0)).astype(jnp.int32)
    idx = jnp.clip(idx, 1, VOCAB - 1)
    kf = idx.astype(jnp.float32)
    dist_lower = jnp.abs(rank_est - kf)
    dist_upper = jnp.abs((kf + jnp.float32(1.0)) - rank_est)
    chosen = jnp.where(dist_upper < dist_lower, idx, idx - 1)
    out_ref[...] = chosen


def _decode_tc(embeddings, proj_pinv, log_v):
    b, _, d = embeddings.shape
    n = b * _HEAD
    return pl.pallas_call(
        _decode_block,
        grid=(1,),
        in_specs=[
            pl.BlockSpec((b, _HEAD, d), lambda i: (0, 0, 0)),
            pl.BlockSpec((proj_pinv.shape[0], 2 * NH), lambda i: (0, 0)),
            pl.BlockSpec(memory_space=pltpu.SMEM),
        ],
        out_specs=pl.BlockSpec((n, 1), lambda i: (0, 0)),
        out_shape=jax.ShapeDtypeStruct((n, 1), jnp.int32),
    )(embeddings, proj_pinv, log_v)


def _sc_body(rank_hbm, c_hbm, out_hbm, rank_v, idx_v, val_v, c_v, tok_v, c16_v,
             inv_sh, sem):
    cid = lax.axis_index("c")
    sid = lax.axis_index("s")
    wid = sid * _NC + cid

    r0 = sid * _VROWS_PER_S
    pltpu.sync_copy(rank_hbm.at[pl.ds(r0, _VROWS_PER_S)], rank_v)

    def conv_row(j, _):
        base = (r0 + j) * 128
        for k in range(8):
            r16 = rank_v[j, pl.ds(k * 16, 16)]
            idx_v[j, pl.ds(k * 16, 16)] = r16.astype(jnp.int32) - 1
            val_v[j, pl.ds(k * 16, 16)] = lax.iota(jnp.int32, 16) + (
                base + k * 16
            )
        return 0

    lax.fori_loop(0, _VROWS_PER_S, conv_row, 0)

    handles = [
        pltpu.async_copy(val_v.at[j], inv_sh.at[idx_v.at[j]], sem)
        for j in range(_VROWS_PER_S)
    ]
    for h in handles:
        h.wait()

    plsc.subcore_barrier()

    p0 = wid * _PROWS_PER_W

    def zero_row(j, _):
        for k in range(8):
            c_v[j, pl.ds(k * 16, 16)] = jnp.zeros((16,), jnp.int32)
        return 0

    lax.fori_loop(0, _PROWS_PER_W, zero_row, 0)
    seq = wid // 8

    @pl.when(wid % 8 == 0)
    def _():
        pltpu.sync_copy(c_hbm.at[pl.ds(seq * _HEAD, _HEAD)], c16_v)
        c_v[0, pl.ds(0, _HEAD)] = c16_v[...]

    handles = [
        pltpu.async_copy(inv_sh.at[c_v.at[j]], tok_v.at[j], sem)
        for j in range(_PROWS_PER_W)
    ]
    for h in handles:
        h.wait()
    pltpu.sync_copy(tok_v, out_hbm.at[pl.ds(p0, _PROWS_PER_W)])


@functools.lru_cache(maxsize=None)
def _decode_sc():
    return pl.kernel(
        _sc_body,
        out_type=jax.ShapeDtypeStruct((_PROWS, 128), jnp.int32),
        mesh=plsc.VectorSubcoreMesh(core_axis_name="c", subcore_axis_name="s"),
        scratch_types=[
            pltpu.VMEM((_VROWS_PER_S, 128), jnp.float32),
            pltpu.VMEM((_VROWS_PER_S, 128), jnp.int32),
            pltpu.VMEM((_VROWS_PER_S, 128), jnp.int32),
            pltpu.VMEM((_PROWS_PER_W, 128), jnp.int32),
            pltpu.VMEM((_PROWS_PER_W, 128), jnp.int32),
            pltpu.VMEM((_HEAD,), jnp.int32),
            pltpu.VMEM_SHARED((_VP,), jnp.int32),
            pltpu.SemaphoreType.DMA,
        ],
    )


def kernel(embeddings, rank_table, proj_weight):
    gram = proj_weight @ proj_weight.T
    proj_pinv = jnp.linalg.solve(gram, proj_weight).T
    log_v = jnp.log(jnp.asarray(float(VOCAB), dtype=jnp.float32)).reshape(1)

    b, seq_t, d = embeddings.shape

    c_head = _decode_tc(embeddings, proj_pinv, log_v).reshape(b * _HEAD)

    pad = jnp.arange(VOCAB + 1, _VP + 1, dtype=jnp.float32)
    rank_p = jnp.concatenate([rank_table, pad]).reshape(_VROWS, 128)
    tok = _decode_sc()(rank_p, c_head)
    return tok.reshape(b, seq_t)


# speedup vs baseline: 1.2744x; 1.2744x over previous
import functools

import jax
import jax.numpy as jnp
import numpy as np
from jax import lax
from jax.experimental import pallas as pl
from jax.experimental.pallas import tpu as pltpu
from jax.experimental.pallas import tpu_sc as plsc

VOCAB = 50257
NH = 2
EPS = 1e-6
TWO_PI = np.float32(2.0 * np.pi)

_NC = 2
_NS = 16
_NW = _NC * _NS

_VROWS = 512
_VP = _VROWS * 128
_VROWS_PER_S = _VROWS // _NS

_PROWS = 256
_PROWS_PER_W = _PROWS // _NW

_HEAD = 16


def _decode_block(emb_ref, pinv_ref, logv_ref, out_ref):
    x = emb_ref[...].reshape(-1, emb_ref.shape[2])
    w = pinv_ref[...]
    sincos = jnp.dot(x, w, preferred_element_type=jnp.float32)
    sin_part = sincos[:, :NH]
    cos_part = sincos[:, NH:]
    phase = jnp.arctan2(sin_part, cos_part)
    n = x.shape[0]
    pos = (lax.broadcasted_iota(jnp.int32, (n, 1), 0) % _HEAD).astype(jnp.float32)
    posc = jnp.maximum(pos, jnp.float32(EPS))
    f1 = phase[:, 0:1] / (TWO_PI * posc)
    f2 = phase[:, 1:2] / ((TWO_PI * np.float32(2.0)) * posc)
    freq_norm = (f1 + f2) * jnp.float32(1.0 / NH)
    freq_norm = jnp.clip(freq_norm, jnp.float32(0.0), jnp.float32(1.0))
    log_v = logv_ref[0]
    rank_est = jnp.exp(freq_norm * log_v)
    idx = jnp.ceil(rank_est - jnp.float32(1.0)).astype(jnp.int32)
    idx = jnp.clip(idx, 1, VOCAB - 1)
    kf = idx.astype(jnp.float32)
    dist_lower = jnp.abs(rank_est - kf)
    dist_upper = jnp.abs((kf + jnp.float32(1.0)) - rank_est)
    chosen = jnp.where(dist_upper < dist_lower, idx, idx - 1)
    out_ref[...] = chosen


def _decode_tc(embeddings, proj_pinv, log_v):
    b, _, d = embeddings.shape
    n = b * _HEAD
    return pl.pallas_call(
        _decode_block,
        grid=(1,),
        in_specs=[
            pl.BlockSpec((b, _HEAD, d), lambda i: (0, 0, 0)),
            pl.BlockSpec((proj_pinv.shape[0], 2 * NH), lambda i: (0, 0)),
            pl.BlockSpec(memory_space=pltpu.SMEM),
        ],
        out_specs=pl.BlockSpec((n, 1), lambda i: (0, 0)),
        out_shape=jax.ShapeDtypeStruct((n, 1), jnp.int32),
    )(embeddings, proj_pinv, log_v)


def _sc_body(rank_hbm, c_hbm, out_hbm, rank_v, idx_v, val_v, tok_v, c16_v,
             tok16_v, t16_v, inv_sh, sem):
    cid = lax.axis_index("c")
    sid = lax.axis_index("s")
    wid = sid * _NC + cid

    r0 = sid * _VROWS_PER_S
    pltpu.sync_copy(rank_hbm.at[pl.ds(r0, _VROWS_PER_S)], rank_v)

    def conv_row(j, _):
        base = (r0 + j) * 128
        for k in range(8):
            r16 = rank_v[j, pl.ds(k * 16, 16)]
            idx_v[j, pl.ds(k * 16, 16)] = r16.astype(jnp.int32) - 1
            val_v[j, pl.ds(k * 16, 16)] = lax.iota(jnp.int32, 16) + (
                base + k * 16
            )
        return 0

    lax.fori_loop(0, _VROWS_PER_S, conv_row, 0)

    handles = [
        pltpu.async_copy(val_v.at[j], inv_sh.at[idx_v.at[j]], sem)
        for j in range(_VROWS_PER_S)
    ]
    for h in handles:
        h.wait()

    plsc.subcore_barrier()

    p0 = wid * _PROWS_PER_W
    pltpu.sync_copy(inv_sh.at[pl.ds(0, 16)], t16_v)
    tok0 = t16_v[pl.ds(0, 16)][0]

    def splat_row(j, _):
        for k in range(8):
            tok_v[j, pl.ds(k * 16, 16)] = jnp.full((16,), tok0, jnp.int32)
        return 0

    lax.fori_loop(0, _PROWS_PER_W, splat_row, 0)

    seq = wid // 8

    @pl.when(wid % 8 == 0)
    def _():
        pltpu.sync_copy(c_hbm.at[pl.ds(seq * _HEAD, _HEAD)], c16_v)
        pltpu.async_copy(inv_sh.at[c16_v], tok16_v, sem).wait()
        tok_v[0, pl.ds(0, _HEAD)] = tok16_v[...]

    pltpu.sync_copy(tok_v, out_hbm.at[pl.ds(p0, _PROWS_PER_W)])


@functools.lru_cache(maxsize=None)
def _decode_sc():
    return pl.kernel(
        _sc_body,
        out_type=jax.ShapeDtypeStruct((_PROWS, 128), jnp.int32),
        mesh=plsc.VectorSubcoreMesh(core_axis_name="c", subcore_axis_name="s"),
        scratch_types=[
            pltpu.VMEM((_VROWS_PER_S, 128), jnp.float32),
            pltpu.VMEM((_VROWS_PER_S, 128), jnp.int32),
            pltpu.VMEM((_VROWS_PER_S, 128), jnp.int32),
            pltpu.VMEM((_PROWS_PER_W, 128), jnp.int32),
            pltpu.VMEM((_HEAD,), jnp.int32),
            pltpu.VMEM((_HEAD,), jnp.int32),
            pltpu.VMEM((16,), jnp.int32),
            pltpu.VMEM_SHARED((_VP,), jnp.int32),
            pltpu.SemaphoreType.DMA,
        ],
    )


def kernel(embeddings, rank_table, proj_weight):
    gram = proj_weight @ proj_weight.T
    proj_pinv = jnp.linalg.solve(gram, proj_weight).T
    log_v = jnp.log(jnp.asarray(float(VOCAB), dtype=jnp.float32)).reshape(1)

    b, seq_t, d = embeddings.shape

    c_head = _decode_tc(embeddings, proj_pinv, log_v).reshape(b * _HEAD)

    pad = jnp.arange(VOCAB + 1, _VP + 1, dtype=jnp.float32)
    rank_p = jnp.concatenate([rank_table, pad]).reshape(_VROWS, 128)
    tok = _decode_sc()(rank_p, c_head)
    return tok.reshape(b, seq_t)
